# Initial kernel scaffold; baseline (speedup 1.0000x reference)
import jax
import jax.numpy as jnp
from jax.experimental import pallas as pl


def kernel(input_1, T_out, T_indices, W1, b1, W2, b2, W3, b3, W4, b4):
    raise NotImplementedError("write your pallas kernel here")


# speedup vs baseline: 1.2049x; 1.2049x over previous
import jax
import jax.numpy as jnp
from jax.experimental import pallas as pl

H = 1000
W = 1000
ROWS_PER_BLK = 8


BLK = ROWS_PER_BLK * W
NBLK = (H * W) // BLK


def _mlp_body(x_ref, w1_ref, b1_ref, w2_ref, b2_ref, w3_ref, b3_ref,
              w4_ref, b4_ref, o_ref):
    x = x_ref[0]
    h = jnp.maximum(jnp.dot(w1_ref[...], x, preferred_element_type=jnp.float32)
                    + b1_ref[...][:, None], 0.0)
    h = jnp.maximum(jnp.dot(w2_ref[...], h, preferred_element_type=jnp.float32)
                    + b2_ref[...][:, None], 0.0)
    h = jnp.maximum(jnp.dot(w3_ref[...], h, preferred_element_type=jnp.float32)
                    + b3_ref[...][:, None], 0.0)
    h = jnp.dot(w4_ref[...], h, preferred_element_type=jnp.float32) \
        + b4_ref[...][:, None]
    o_ref[0] = h


def _mlp(x, W1, b1, W2, b2, W3, b3, W4, b4):
    full = lambda s: pl.BlockSpec(s, lambda i: (0,) * len(s))
    return pl.pallas_call(
        _mlp_body,
        grid=(NBLK,),
        in_specs=[
            pl.BlockSpec((1, 7, BLK), lambda i: (i, 0, 0)),
            full((18, 7)), full((18,)),
            full((36, 18)), full((36,)),
            full((36, 36)), full((36,)),
            full((1, 36)), full((1,)),
        ],
        out_specs=pl.BlockSpec((1, 1, BLK), lambda i: (i, 0, 0)),
        out_shape=jax.ShapeDtypeStruct((NBLK, 1, BLK), jnp.float32),
    )(x, W1, b1, W2, b2, W3, b3, W4, b4)


def kernel(input_1, T_out, T_indices, W1, b1, W2, b2, W3, b3, W4, b4):
    x = input_1[0].reshape(7, NBLK, BLK).transpose(1, 0, 2)
    x0 = _mlp(x, W1, b1, W2, b2, W3, b3, W4, b4)
    v = x0.reshape(-1)
    keys = (T_indices[0].astype(jnp.int32) * W
            + T_indices[1].astype(jnp.int32)).reshape(-1)
    n = jnp.arange(H * W, dtype=jnp.int32)
    win = jnp.full((H * W,), -1, jnp.int32).at[keys].max(n)
    grid_vals = jnp.where(win >= 0, v[jnp.maximum(win, 0)], -9999.0)
    grid_vals = grid_vals.reshape(H, W)
    x1 = jnp.max(grid_vals, axis=1)
    x2 = jnp.max(grid_vals, axis=0)
    return (x1, x2)


# speedup vs baseline: 3.3418x; 2.7736x over previous
import functools

import jax
import jax.numpy as jnp
from jax import lax
from jax.experimental import pallas as pl
from jax.experimental.pallas import tpu as pltpu
from jax.experimental.pallas import tpu_sc as plsc

H = 1000
W = 1000
HW = H * W


BLK = 8000
NBLK = HW // BLK

ROW_STRIDE = 1024
GRID_ROWS_PER_WORKER = 32
NWORKERS = 32
GRID_WORDS = GRID_ROWS_PER_WORKER * ROW_STRIDE

WINDOW = 4000
NWIN = HW // WINDOW
VPW = WINDOW // 16


def _mlp_body(x_ref, idx_ref, w1_ref, b1_ref, w2_ref, b2_ref, w3_ref, b3_ref,
              w4_ref, b4_ref, o_ref, k_ref):
    x = x_ref[0]
    h = jnp.maximum(jnp.dot(w1_ref[...], x, preferred_element_type=jnp.float32)
                    + b1_ref[...][:, None], 0.0)
    h = jnp.maximum(jnp.dot(w2_ref[...], h, preferred_element_type=jnp.float32)
                    + b2_ref[...][:, None], 0.0)
    h = jnp.maximum(jnp.dot(w3_ref[...], h, preferred_element_type=jnp.float32)
                    + b3_ref[...][:, None], 0.0)
    h = jnp.dot(w4_ref[...], h, preferred_element_type=jnp.float32) \
        + b4_ref[...][:, None]
    o_ref[0] = h
    idx = idx_ref[0]
    k_ref[0] = idx[0:1, :] * ROW_STRIDE + idx[1:2, :]


def _mlp(x, t_idx, W1, b1, W2, b2, W3, b3, W4, b4):
    full = lambda s: pl.BlockSpec(s, lambda i: (0,) * len(s))
    return pl.pallas_call(
        _mlp_body,
        grid=(NBLK,),
        in_specs=[
            pl.BlockSpec((1, 7, BLK), lambda i: (i, 0, 0)),
            pl.BlockSpec((1, 2, BLK), lambda i: (i, 0, 0)),
            full((18, 7)), full((18,)),
            full((36, 18)), full((36,)),
            full((36, 36)), full((36,)),
            full((1, 36)), full((1,)),
        ],
        out_specs=[
            pl.BlockSpec((1, 1, BLK), lambda i: (i, 0, 0)),
            pl.BlockSpec((1, 1, BLK), lambda i: (i, 0, 0)),
        ],
        out_shape=[
            jax.ShapeDtypeStruct((NBLK, 1, BLK), jnp.float32),
            jax.ShapeDtypeStruct((NBLK, 1, BLK), jnp.int32),
        ],
    )(x, t_idx, W1, b1, W2, b2, W3, b3, W4, b4)


def _gather16(a, idx):
    return lax.gather(
        a, idx[:, None],
        dimension_numbers=lax.GatherDimensionNumbers(
            offset_dims=(), collapsed_slice_dims=(0,), start_index_map=(0,)),
        slice_sizes=(1,),
        mode=lax.GatherScatterMode.PROMISE_IN_BOUNDS)


def _sc_body(key_hbm, val_hbm, rowmax_hbm, colmax_hbm,
             grid_v, kbuf0, kbuf1, vbuf0, vbuf1, rowmax_v, colmax_v,
             semk0, semk1, semv0, semv1):
    kbuf = (kbuf0, kbuf1)
    vbuf = (vbuf0, vbuf1)
    wid = lax.axis_index("s") * 2 + lax.axis_index("c")
    lo = wid * GRID_WORDS
    hi = lo + GRID_WORDS
    lane = lax.iota(jnp.int32, 16)
    neg = jnp.full((16,), -9999.0, jnp.float32)
    semk = (semk0, semk1)
    semv = (semv0, semv1)

    def init_body(i, _):
        grid_v[pl.ds(i * 16, 16)] = neg
        return 0
    lax.fori_loop(0, GRID_WORDS // 16, init_body, 0, unroll=8)

    def start_win(win, b):
        pltpu.async_copy(key_hbm.at[pl.ds(win * WINDOW, WINDOW)],
                         kbuf[b], semk[b])
        pltpu.async_copy(val_hbm.at[pl.ds(win * WINDOW, WINDOW)],
                         vbuf[b], semv[b])

    def wait_win(b):
        pltpu.make_async_copy(key_hbm.at[pl.ds(0, WINDOW)],
                              kbuf[b], semk[b]).wait()
        pltpu.make_async_copy(val_hbm.at[pl.ds(0, WINDOW)],
                              vbuf[b], semv[b]).wait()

    def process(b):
        kb = kbuf[b]
        vb = vbuf[b]

        def vloop(j, _):
            kk = kb[pl.ds(j * 16, 16)]
            vv = vb[pl.ds(j * 16, 16)]
            sk, perm = plsc.sort_key_val(kk * 16 + lane, lane)
            kk_s = lax.shift_right_logical(sk, 4)
            nxt = _gather16(kk_s, jnp.minimum(lane + 1, 15))
            winner = (kk_s != nxt) | (lane == 15)
            inr = (kk_s >= lo) & (kk_s < hi)
            vs = _gather16(vv, perm)
            plsc.store_scatter(grid_v, [kk_s - lo], vs, mask=winner & inr)
            return 0
        lax.fori_loop(0, VPW, vloop, 0)

    start_win(0, 0)
    start_win(1, 1)

    def outer(i, _):
        for b in range(2):
            win = 2 * i + b
            wait_win(b)
            process(b)

            @pl.when(win + 2 < NWIN)
            def _():
                start_win(win + 2, b)
        return 0
    lax.fori_loop(0, NWIN // 2, outer, 0)

    def rowred(lr, _):
        def fold(cg, acc):
            return jnp.maximum(acc, grid_v[pl.ds(lr * ROW_STRIDE + cg * 16, 16)])
        acc = lax.fori_loop(0, ROW_STRIDE // 16, fold, neg, unroll=8)
        m = jnp.max(acc)
        plsc.store_scatter(rowmax_v, [jnp.full((16,), lr, jnp.int32)],
                           jnp.full((16,), m, jnp.float32), mask=(lane == 0))
        return 0
    lax.fori_loop(0, GRID_ROWS_PER_WORKER, rowred, 0)

    def colred(cg, _):
        def fold(rr, acc):
            return jnp.maximum(acc, grid_v[pl.ds(rr * ROW_STRIDE + cg * 16, 16)])
        acc = lax.fori_loop(0, GRID_ROWS_PER_WORKER, fold, neg, unroll=8)
        colmax_v[pl.ds(cg * 16, 16)] = acc
        return 0
    lax.fori_loop(0, ROW_STRIDE // 16, colred, 0)

    pltpu.sync_copy(rowmax_v, rowmax_hbm.at[pl.ds(wid * GRID_ROWS_PER_WORKER,
                                                  GRID_ROWS_PER_WORKER)])
    pltpu.sync_copy(colmax_v, colmax_hbm.at[wid])


def _sc_scatter_max(key_flat, val_flat):
    mesh = plsc.VectorSubcoreMesh(core_axis_name="c", subcore_axis_name="s")
    call = pl.kernel(
        _sc_body,
        out_type=[
            jax.ShapeDtypeStruct((NWORKERS * GRID_ROWS_PER_WORKER,), jnp.float32),
            jax.ShapeDtypeStruct((NWORKERS, ROW_STRIDE), jnp.float32),
        ],
        mesh=mesh,
        compiler_params=pltpu.CompilerParams(needs_layout_passes=False),
        scratch_types=[
            pltpu.VMEM((GRID_WORDS,), jnp.float32),
            pltpu.VMEM((WINDOW,), jnp.int32),
            pltpu.VMEM((WINDOW,), jnp.int32),
            pltpu.VMEM((WINDOW,), jnp.float32),
            pltpu.VMEM((WINDOW,), jnp.float32),
            pltpu.VMEM((GRID_ROWS_PER_WORKER,), jnp.float32),
            pltpu.VMEM((ROW_STRIDE,), jnp.float32),
            pltpu.SemaphoreType.DMA,
            pltpu.SemaphoreType.DMA,
            pltpu.SemaphoreType.DMA,
            pltpu.SemaphoreType.DMA,
        ],
    )
    return call(key_flat, val_flat)


def kernel(input_1, T_out, T_indices, W1, b1, W2, b2, W3, b3, W4, b4):
    x = input_1[0].reshape(7, NBLK, BLK).transpose(1, 0, 2)
    t_idx = T_indices.reshape(2, NBLK, BLK).transpose(1, 0, 2)
    v, k = _mlp(x, t_idx, W1, b1, W2, b2, W3, b3, W4, b4)
    rowmax, colmax = _sc_scatter_max(k.reshape(-1), v.reshape(-1))
    x1 = rowmax[:H]
    x2 = jnp.max(colmax, axis=0)[:W]
    return (x1, x2)


# speedup vs baseline: 6.8665x; 2.0547x over previous
import functools

import jax
import jax.numpy as jnp
from jax import lax
from jax.experimental import pallas as pl
from jax.experimental.pallas import tpu as pltpu
from jax.experimental.pallas import tpu_sc as plsc

H = 1000
W = 1000
HW = H * W


BLK = 8000
NBLK = HW // BLK

ROW_STRIDE = 1024
GRID_ROWS_PER_WORKER = 32
NWORKERS = 32
GRID_WORDS = GRID_ROWS_PER_WORKER * ROW_STRIDE

WINDOW = 4000
NWIN = HW // WINDOW
VPW = WINDOW // 16


def _mlp_body(x_ref, idx_ref, w1_ref, b1_ref, w2_ref, b2_ref, w3_ref, b3_ref,
              w4_ref, b4_ref, o_ref, k_ref):
    x = x_ref[0]
    h = jnp.maximum(jnp.dot(w1_ref[...], x, preferred_element_type=jnp.float32)
                    + b1_ref[...][:, None], 0.0)
    h = jnp.maximum(jnp.dot(w2_ref[...], h, preferred_element_type=jnp.float32)
                    + b2_ref[...][:, None], 0.0)
    h = jnp.maximum(jnp.dot(w3_ref[...], h, preferred_element_type=jnp.float32)
                    + b3_ref[...][:, None], 0.0)
    h = jnp.dot(w4_ref[...], h, preferred_element_type=jnp.float32) \
        + b4_ref[...][:, None]
    o_ref[0] = h
    idx = idx_ref[0]
    k_ref[0] = idx[0:1, :] * ROW_STRIDE + idx[1:2, :]


def _mlp(x, t_idx, W1, b1, W2, b2, W3, b3, W4, b4):
    full = lambda s: pl.BlockSpec(s, lambda i: (0,) * len(s))
    return pl.pallas_call(
        _mlp_body,
        grid=(NBLK,),
        in_specs=[
            pl.BlockSpec((1, 7, BLK), lambda i: (i, 0, 0)),
            pl.BlockSpec((1, 2, BLK), lambda i: (i, 0, 0)),
            full((18, 7)), full((18,)),
            full((36, 18)), full((36,)),
            full((36, 36)), full((36,)),
            full((1, 36)), full((1,)),
        ],
        out_specs=[
            pl.BlockSpec((1, 1, BLK), lambda i: (i, 0, 0)),
            pl.BlockSpec((1, 1, BLK), lambda i: (i, 0, 0)),
        ],
        out_shape=[
            jax.ShapeDtypeStruct((NBLK, 1, BLK), jnp.float32),
            jax.ShapeDtypeStruct((NBLK, 1, BLK), jnp.int32),
        ],
    )(x, t_idx, W1, b1, W2, b2, W3, b3, W4, b4)


def _gather16(a, idx):
    return lax.gather(
        a, idx[:, None],
        dimension_numbers=lax.GatherDimensionNumbers(
            offset_dims=(), collapsed_slice_dims=(0,), start_index_map=(0,)),
        slice_sizes=(1,),
        mode=lax.GatherScatterMode.PROMISE_IN_BOUNDS)


def _sc_body(key_hbm, val_hbm, rowmax_hbm, colmax_hbm,
             grid_v, kbuf0, kbuf1, vbuf0, vbuf1, rowmax_v, colmax_v,
             semk0, semk1, semv0, semv1):
    kbuf = (kbuf0, kbuf1)
    vbuf = (vbuf0, vbuf1)
    wid = lax.axis_index("s") * 2 + lax.axis_index("c")
    lo = wid * GRID_WORDS
    hi = lo + GRID_WORDS
    lane = lax.iota(jnp.int32, 16)
    neg = jnp.full((16,), -9999.0, jnp.float32)
    semk = (semk0, semk1)
    semv = (semv0, semv1)

    def init_body(i, _):
        grid_v[pl.ds(i * 16, 16)] = neg
        return 0
    lax.fori_loop(0, GRID_WORDS // 16, init_body, 0, unroll=8)

    def start_win(win, b):
        pltpu.async_copy(key_hbm.at[pl.ds(win * WINDOW, WINDOW)],
                         kbuf[b], semk[b])
        pltpu.async_copy(val_hbm.at[pl.ds(win * WINDOW, WINDOW)],
                         vbuf[b], semv[b])

    def wait_win(b):
        pltpu.make_async_copy(key_hbm.at[pl.ds(0, WINDOW)],
                              kbuf[b], semk[b]).wait()
        pltpu.make_async_copy(val_hbm.at[pl.ds(0, WINDOW)],
                              vbuf[b], semv[b]).wait()

    def process(b):
        kb = kbuf[b]
        vb = vbuf[b]

        def vloop(j, _):
            kk = kb[pl.ds(j * 16, 16)]
            vv = vb[pl.ds(j * 16, 16)]
            inr = (kk >= lo) & (kk < hi)
            plsc.store_scatter(grid_v, [kk - lo], vv, mask=inr)
            return 0
        lax.fori_loop(0, VPW, vloop, 0, unroll=8)

    start_win(0, 0)
    start_win(1, 1)

    def outer(i, _):
        for b in range(2):
            win = 2 * i + b
            wait_win(b)
            process(b)

            @pl.when(win + 2 < NWIN)
            def _():
                start_win(win + 2, b)
        return 0
    lax.fori_loop(0, NWIN // 2, outer, 0)

    def rowred(lr, _):
        def fold(cg, acc):
            return jnp.maximum(acc, grid_v[pl.ds(lr * ROW_STRIDE + cg * 16, 16)])
        acc = lax.fori_loop(0, ROW_STRIDE // 16, fold, neg, unroll=8)
        m = jnp.max(acc)
        plsc.store_scatter(rowmax_v, [jnp.full((16,), lr, jnp.int32)],
                           jnp.full((16,), m, jnp.float32), mask=(lane == 0))
        return 0
    lax.fori_loop(0, GRID_ROWS_PER_WORKER, rowred, 0)

    def colred(cg, _):
        def fold(rr, acc):
            return jnp.maximum(acc, grid_v[pl.ds(rr * ROW_STRIDE + cg * 16, 16)])
        acc = lax.fori_loop(0, GRID_ROWS_PER_WORKER, fold, neg, unroll=8)
        colmax_v[pl.ds(cg * 16, 16)] = acc
        return 0
    lax.fori_loop(0, ROW_STRIDE // 16, colred, 0)

    pltpu.sync_copy(rowmax_v, rowmax_hbm.at[pl.ds(wid * GRID_ROWS_PER_WORKER,
                                                  GRID_ROWS_PER_WORKER)])
    pltpu.sync_copy(colmax_v, colmax_hbm.at[wid])


def _sc_scatter_max(key_flat, val_flat):
    mesh = plsc.VectorSubcoreMesh(core_axis_name="c", subcore_axis_name="s")
    call = pl.kernel(
        _sc_body,
        out_type=[
            jax.ShapeDtypeStruct((NWORKERS * GRID_ROWS_PER_WORKER,), jnp.float32),
            jax.ShapeDtypeStruct((NWORKERS, ROW_STRIDE), jnp.float32),
        ],
        mesh=mesh,
        compiler_params=pltpu.CompilerParams(needs_layout_passes=False),
        scratch_types=[
            pltpu.VMEM((GRID_WORDS,), jnp.float32),
            pltpu.VMEM((WINDOW,), jnp.int32),
            pltpu.VMEM((WINDOW,), jnp.int32),
            pltpu.VMEM((WINDOW,), jnp.float32),
            pltpu.VMEM((WINDOW,), jnp.float32),
            pltpu.VMEM((GRID_ROWS_PER_WORKER,), jnp.float32),
            pltpu.VMEM((ROW_STRIDE,), jnp.float32),
            pltpu.SemaphoreType.DMA,
            pltpu.SemaphoreType.DMA,
            pltpu.SemaphoreType.DMA,
            pltpu.SemaphoreType.DMA,
        ],
    )
    return call(key_flat, val_flat)


def kernel(input_1, T_out, T_indices, W1, b1, W2, b2, W3, b3, W4, b4):
    x = input_1[0].reshape(7, NBLK, BLK).transpose(1, 0, 2)
    t_idx = T_indices.reshape(2, NBLK, BLK).transpose(1, 0, 2)
    v, k = _mlp(x, t_idx, W1, b1, W2, b2, W3, b3, W4, b4)
    rowmax, colmax = _sc_scatter_max(k.reshape(-1), v.reshape(-1))
    x1 = rowmax[:H]
    x2 = jnp.max(colmax, axis=0)[:W]
    return (x1, x2)


# speedup vs baseline: 14.0051x; 2.0396x over previous
import functools

import jax
import jax.numpy as jnp
from jax import lax
from jax.experimental import pallas as pl
from jax.experimental.pallas import tpu as pltpu
from jax.experimental.pallas import tpu_sc as plsc

H = 1000
W = 1000
HW = H * W


BLK = 8000
NBLK = HW // BLK

ROW_STRIDE = 1024
GRID_ROWS_PER_WORKER = 32
NWORKERS = 32
GRID_WORDS = GRID_ROWS_PER_WORKER * ROW_STRIDE

CHUNK = 32000
LAST_CHUNK = HW - 31 * CHUNK
CAP = 1536
WINDOW = 2000
NWIN_FULL = CHUNK // WINDOW
NWIN_LAST = LAST_CHUNK // WINDOW
VPW = WINDOW // 16


def _mlp_body(x_ref, idx_ref, w1_ref, b1_ref, w2_ref, b2_ref, w3_ref, b3_ref,
              w4_ref, b4_ref, o_ref, k_ref):
    x = x_ref[0]
    h = jnp.maximum(jnp.dot(w1_ref[...], x, preferred_element_type=jnp.float32)
                    + b1_ref[...][:, None], 0.0)
    h = jnp.maximum(jnp.dot(w2_ref[...], h, preferred_element_type=jnp.float32)
                    + b2_ref[...][:, None], 0.0)
    h = jnp.maximum(jnp.dot(w3_ref[...], h, preferred_element_type=jnp.float32)
                    + b3_ref[...][:, None], 0.0)
    h = jnp.dot(w4_ref[...], h, preferred_element_type=jnp.float32) \
        + b4_ref[...][:, None]
    o_ref[0] = h
    idx = idx_ref[0]
    k_ref[0] = idx[0:1, :] * ROW_STRIDE + idx[1:2, :]


def _mlp(x, t_idx, W1, b1, W2, b2, W3, b3, W4, b4):
    full = lambda s: pl.BlockSpec(s, lambda i: (0,) * len(s))
    return pl.pallas_call(
        _mlp_body,
        grid=(NBLK,),
        in_specs=[
            pl.BlockSpec((1, 7, BLK), lambda i: (i, 0, 0)),
            pl.BlockSpec((1, 2, BLK), lambda i: (i, 0, 0)),
            full((18, 7)), full((18,)),
            full((36, 18)), full((36,)),
            full((36, 36)), full((36,)),
            full((1, 36)), full((1,)),
        ],
        out_specs=[
            pl.BlockSpec((1, 1, BLK), lambda i: (i, 0, 0)),
            pl.BlockSpec((1, 1, BLK), lambda i: (i, 0, 0)),
        ],
        out_shape=[
            jax.ShapeDtypeStruct((NBLK, 1, BLK), jnp.float32),
            jax.ShapeDtypeStruct((NBLK, 1, BLK), jnp.int32),
        ],
    )(x, t_idx, W1, b1, W2, b2, W3, b3, W4, b4)


def _gather16(a, idx):
    return lax.gather(
        a, idx[:, None],
        dimension_numbers=lax.GatherDimensionNumbers(
            offset_dims=(), collapsed_slice_dims=(0,), start_index_map=(0,)),
        slice_sizes=(1,),
        mode=lax.GatherScatterMode.PROMISE_IN_BOUNDS)


def _route_body(key_hbm, val_hbm, bk_hbm, bv_hbm, cnt_hbm,
                kstage, vstage, kbuf0, kbuf1, vbuf0, vbuf1, cursor_v,
                semk0, semk1, semv0, semv1, semout):
    kbuf = (kbuf0, kbuf1)
    vbuf = (vbuf0, vbuf1)
    semk = (semk0, semk1)
    semv = (semv0, semv1)
    wid = lax.axis_index("s") * 2 + lax.axis_index("c")
    base = wid * CHUNK
    nwin = jnp.where(wid == NWORKERS - 1, NWIN_LAST, NWIN_FULL)
    lane = lax.iota(jnp.int32, 16)
    ones = jnp.ones((16,), jnp.int32)

    cursor_v[pl.ds(0, 16)] = jnp.zeros((16,), jnp.int32)
    cursor_v[pl.ds(16, 16)] = jnp.zeros((16,), jnp.int32)

    def start_win(win, b):
        off = base + win * WINDOW
        pltpu.async_copy(key_hbm.at[pl.ds(off, WINDOW)], kbuf[b], semk[b])
        pltpu.async_copy(val_hbm.at[pl.ds(off, WINDOW)], vbuf[b], semv[b])

    def wait_win(b):
        pltpu.make_async_copy(key_hbm.at[pl.ds(0, WINDOW)],
                              kbuf[b], semk[b]).wait()
        pltpu.make_async_copy(val_hbm.at[pl.ds(0, WINDOW)],
                              vbuf[b], semv[b]).wait()

    def process(b):
        kb = kbuf[b]
        vb = vbuf[b]

        def vloop(j, _):
            kk = kb[pl.ds(j * 16, 16)]
            vv = vb[pl.ds(j * 16, 16)]
            d = lax.shift_right_logical(kk, 15)
            s, perm = plsc.sort_key_val(d * 16 + lane, lane)
            d_s = lax.shift_right_logical(s, 4)
            prev = _gather16(d_s, jnp.maximum(lane - 1, 0))
            is_start = (lane == 0) | (d_s != prev)
            start_pos = plsc.cummax(jnp.where(is_start, lane, 0))
            rank = lane - start_pos
            old = plsc.load_gather(cursor_v, [d_s])
            pos = old + rank
            addr = d_s * CAP + pos
            k_s = _gather16(kk, perm)
            v_s = _gather16(vv, perm)
            plsc.store_scatter(kstage, [addr], k_s)
            plsc.store_scatter(vstage, [addr], v_s)
            plsc.addupdate_scatter(cursor_v, [d_s], ones)
            return 0
        lax.fori_loop(0, VPW, vloop, 0, unroll=4)

    start_win(0, 0)
    start_win(1, 1)

    def outer(i, _):
        for b in range(2):
            win = 2 * i + b
            wait_win(b)
            process(b)

            @pl.when(win + 2 < nwin)
            def _():
                start_win(win + 2, b)
        return 0
    lax.fori_loop(0, nwin // 2, outer, 0)

    for d in range(NWORKERS):
        dst = pl.ds((d * NWORKERS + wid.astype(jnp.int32)) * CAP, CAP)
        pltpu.async_copy(kstage.at[pl.ds(d * CAP, CAP)], bk_hbm.at[dst], semout)
        pltpu.async_copy(vstage.at[pl.ds(d * CAP, CAP)], bv_hbm.at[dst], semout)
    for d in range(NWORKERS):
        pltpu.make_async_copy(kstage.at[pl.ds(0, CAP)],
                              bk_hbm.at[pl.ds(0, CAP)], semout).wait()
        pltpu.make_async_copy(vstage.at[pl.ds(0, CAP)],
                              bv_hbm.at[pl.ds(0, CAP)], semout).wait()
    pltpu.sync_copy(cursor_v, cnt_hbm.at[wid])


def _route(key_flat, val_flat):
    mesh = plsc.VectorSubcoreMesh(core_axis_name="c", subcore_axis_name="s")
    call = pl.kernel(
        _route_body,
        out_type=[
            jax.ShapeDtypeStruct((NWORKERS * NWORKERS * CAP,), jnp.int32),
            jax.ShapeDtypeStruct((NWORKERS * NWORKERS * CAP,), jnp.float32),
            jax.ShapeDtypeStruct((NWORKERS, NWORKERS), jnp.int32),
        ],
        mesh=mesh,
        compiler_params=pltpu.CompilerParams(needs_layout_passes=False),
        scratch_types=[
            pltpu.VMEM((NWORKERS * CAP,), jnp.int32),
            pltpu.VMEM((NWORKERS * CAP,), jnp.float32),
            pltpu.VMEM((WINDOW,), jnp.int32),
            pltpu.VMEM((WINDOW,), jnp.int32),
            pltpu.VMEM((WINDOW,), jnp.float32),
            pltpu.VMEM((WINDOW,), jnp.float32),
            pltpu.VMEM((NWORKERS,), jnp.int32),
            pltpu.SemaphoreType.DMA,
            pltpu.SemaphoreType.DMA,
            pltpu.SemaphoreType.DMA,
            pltpu.SemaphoreType.DMA,
            pltpu.SemaphoreType.DMA,
        ],
    )
    return call(key_flat, val_flat)


def _sc_body(bk_hbm, bv_hbm, cnt_hbm, rowmax_hbm, colmax_hbm,
             grid_v, kbuf0, kbuf1, vbuf0, vbuf1, cnt_v, cc_v,
             rowmax_v, colmax_v,
             semk0, semk1, semv0, semv1, semc):
    kbuf = (kbuf0, kbuf1)
    vbuf = (vbuf0, vbuf1)
    semk = (semk0, semk1)
    semv = (semv0, semv1)
    wid = lax.axis_index("s") * 2 + lax.axis_index("c")
    lo = wid * GRID_WORDS
    lane = lax.iota(jnp.int32, 16)
    neg = jnp.full((16,), -9999.0, jnp.float32)

    pltpu.async_copy(cnt_hbm, cnt_v, semc)

    def init_body(i, _):
        grid_v[pl.ds(i * 16, 16)] = neg
        return 0
    lax.fori_loop(0, GRID_WORDS // 16, init_body, 0, unroll=8)

    def start_seg(w, b):
        src = pl.ds((wid * NWORKERS + w) * CAP, CAP)
        pltpu.async_copy(bk_hbm.at[src], kbuf[b], semk[b])
        pltpu.async_copy(bv_hbm.at[src], vbuf[b], semv[b])

    def wait_seg(b):
        pltpu.make_async_copy(bk_hbm.at[pl.ds(0, CAP)],
                              kbuf[b], semk[b]).wait()
        pltpu.make_async_copy(bv_hbm.at[pl.ds(0, CAP)],
                              vbuf[b], semv[b]).wait()

    pltpu.make_async_copy(cnt_hbm, cnt_v, semc).wait()
    cc_v[pl.ds(0, 16)] = plsc.load_gather(cnt_v, [lane * NWORKERS + wid])
    cc_v[pl.ds(16, 16)] = plsc.load_gather(cnt_v,
                                           [(lane + 16) * NWORKERS + wid])

    def process(w, b):
        kb = kbuf[b]
        vb = vbuf[b]
        cvec = cc_v[pl.ds(lax.shift_right_logical(w, 4) * 16, 16)]
        cnt = _gather16(cvec, jnp.full((16,), jnp.bitwise_and(w, 15),
                                       jnp.int32))

        def vloop(j, _):
            kk = kb[pl.ds(j * 16, 16)]
            vv = vb[pl.ds(j * 16, 16)]
            valid = (lane + j * 16) < cnt
            plsc.store_scatter(grid_v, [kk - lo], vv, mask=valid)
            return 0
        lax.fori_loop(0, CAP // 16, vloop, 0, unroll=8)

    start_seg(0, 0)
    start_seg(1, 1)

    def outer(i, _):
        for b in range(2):
            w = 2 * i + b
            wait_seg(b)
            process(w, b)

            @pl.when(w + 2 < NWORKERS)
            def _():
                start_seg(w + 2, b)
        return 0
    lax.fori_loop(0, NWORKERS // 2, outer, 0)

    def rowred(lr, _):
        def fold(cg, acc):
            return jnp.maximum(acc, grid_v[pl.ds(lr * ROW_STRIDE + cg * 16, 16)])
        acc = lax.fori_loop(0, ROW_STRIDE // 16, fold, neg, unroll=8)
        m = jnp.max(acc)
        plsc.store_scatter(rowmax_v, [jnp.full((16,), lr, jnp.int32)],
                           jnp.full((16,), m, jnp.float32), mask=(lane == 0))
        return 0
    lax.fori_loop(0, GRID_ROWS_PER_WORKER, rowred, 0)

    def colred(cg, _):
        def fold(rr, acc):
            return jnp.maximum(acc, grid_v[pl.ds(rr * ROW_STRIDE + cg * 16, 16)])
        acc = lax.fori_loop(0, GRID_ROWS_PER_WORKER, fold, neg, unroll=8)
        colmax_v[pl.ds(cg * 16, 16)] = acc
        return 0
    lax.fori_loop(0, ROW_STRIDE // 16, colred, 0)

    pltpu.sync_copy(rowmax_v, rowmax_hbm.at[pl.ds(wid * GRID_ROWS_PER_WORKER,
                                                  GRID_ROWS_PER_WORKER)])
    pltpu.sync_copy(colmax_v, colmax_hbm.at[wid])


def _sc_scatter_max(bk, bv, cnt):
    mesh = plsc.VectorSubcoreMesh(core_axis_name="c", subcore_axis_name="s")
    call = pl.kernel(
        _sc_body,
        out_type=[
            jax.ShapeDtypeStruct((NWORKERS * GRID_ROWS_PER_WORKER,), jnp.float32),
            jax.ShapeDtypeStruct((NWORKERS, ROW_STRIDE), jnp.float32),
        ],
        mesh=mesh,
        compiler_params=pltpu.CompilerParams(needs_layout_passes=False),
        scratch_types=[
            pltpu.VMEM((GRID_WORDS,), jnp.float32),
            pltpu.VMEM((CAP,), jnp.int32),
            pltpu.VMEM((CAP,), jnp.int32),
            pltpu.VMEM((CAP,), jnp.float32),
            pltpu.VMEM((CAP,), jnp.float32),
            pltpu.VMEM((NWORKERS * NWORKERS,), jnp.int32),
            pltpu.VMEM((NWORKERS,), jnp.int32),
            pltpu.VMEM((GRID_ROWS_PER_WORKER,), jnp.float32),
            pltpu.VMEM((ROW_STRIDE,), jnp.float32),
            pltpu.SemaphoreType.DMA,
            pltpu.SemaphoreType.DMA,
            pltpu.SemaphoreType.DMA,
            pltpu.SemaphoreType.DMA,
            pltpu.SemaphoreType.DMA,
        ],
    )
    return call(bk, bv, cnt)


def kernel(input_1, T_out, T_indices, W1, b1, W2, b2, W3, b3, W4, b4):
    x = input_1[0].reshape(7, NBLK, BLK).transpose(1, 0, 2)
    t_idx = T_indices.reshape(2, NBLK, BLK).transpose(1, 0, 2)
    v, k = _mlp(x, t_idx, W1, b1, W2, b2, W3, b3, W4, b4)
    bk, bv, cnt = _route(k.reshape(-1), v.reshape(-1))
    rowmax, colmax = _sc_scatter_max(bk, bv, cnt.reshape(-1))
    x1 = rowmax[:H]
    x2 = jnp.max(colmax, axis=0)[:W]
    return (x1, x2)
